# baseline (speedup 1.0000x reference)
import functools

import jax
import jax.numpy as jnp
from jax import lax
from jax.experimental import pallas as pl
from jax.experimental.pallas import tpu as pltpu
from jax.experimental.pallas import tpu_sc as plsc

E = 8
K = 2
D = 1024
F = 2048
N = 4096
S = N * K
B = 768
T = (S + E * (B - 1) + B - 1) // B
R = T * B

_NEG = -1e30
_GATE_BN = 512
_LANES = 128


def _gate_body(x_ref, wg_ref, sel_ref, w0_ref, w1_ref):
    logits = jnp.dot(x_ref[...], wg_ref[...], preferred_element_type=jnp.float32)
    col = lax.broadcasted_iota(jnp.int32, logits.shape, 1)
    logits = jnp.where(col < E, logits, _NEG)
    m1 = jnp.max(logits, axis=1, keepdims=True)
    a1 = jnp.min(jnp.where(logits == m1, col, _LANES), axis=1, keepdims=True)
    l2 = jnp.where(col == a1, _NEG, logits)
    m2 = jnp.max(l2, axis=1, keepdims=True)
    a2 = jnp.min(jnp.where(l2 == m2, col, _LANES), axis=1, keepdims=True)
    e2 = jnp.exp(m2 - m1)
    w0 = 1.0 / (1.0 + e2)
    sel_ref[...] = jnp.where(col == 0, a1, jnp.where(col == 1, a2, 0)).astype(jnp.int32)
    w0_ref[...] = jnp.broadcast_to(w0, w0_ref.shape)
    w1_ref[...] = jnp.broadcast_to(1.0 - w0, w1_ref.shape)


def _gate(inputs, Wg):
    wg_pad = jnp.pad(Wg, ((0, 0), (0, _LANES - E)))
    sel_pad, wb0, wb1 = pl.pallas_call(
        _gate_body,
        grid=(N // _GATE_BN,),
        in_specs=[
            pl.BlockSpec((_GATE_BN, D), lambda i: (i, 0)),
            pl.BlockSpec((D, _LANES), lambda i: (0, 0)),
        ],
        out_specs=[
            pl.BlockSpec((_GATE_BN, _LANES), lambda i: (i, 0)),
            pl.BlockSpec((_GATE_BN, _LANES), lambda i: (i, 0)),
            pl.BlockSpec((_GATE_BN, _LANES), lambda i: (i, 0)),
        ],
        out_shape=[
            jax.ShapeDtypeStruct((N, _LANES), jnp.int32),
            jax.ShapeDtypeStruct((N, _LANES), jnp.float32),
            jax.ShapeDtypeStruct((N, _LANES), jnp.float32),
        ],
    )(inputs, wg_pad)
    return sel_pad[:, :K], wb0, wb1


def _routing(sel):
    s = sel.reshape(-1)
    onehot = (s[:, None] == jnp.arange(E, dtype=jnp.int32)[None, :]).astype(jnp.int32)
    csum = jnp.cumsum(onehot, axis=0)
    rank = jnp.take_along_axis(csum, s[:, None], axis=1)[:, 0] - 1
    sizes = csum[-1]
    padded = ((sizes + B - 1) // B) * B
    bounds = jnp.cumsum(padded)
    starts = bounds - padded
    pos = starts[s] + rank
    tile_first = jnp.arange(T, dtype=jnp.int32) * B
    texp = jnp.minimum(
        jnp.searchsorted(bounds, tile_first, side="right"), E - 1
    ).astype(jnp.int32)
    a_tiles = bounds[-1] // B
    ii = jnp.arange(T, dtype=jnp.int32)
    xblk = jnp.minimum(ii, a_tiles - 1)
    act = (ii < a_tiles).astype(jnp.int32)
    sa = jnp.stack([texp[xblk], act, xblk, jnp.zeros_like(ii)], axis=1)
    return pos, sa


def _mlp_body(sa_ref, x_ref, w1_ref, w2_ref, wd_ref, y_ref):
    i = pl.program_id(0)

    @pl.when(sa_ref[i, 1] != 0)
    def _():
        h = jnp.dot(x_ref[...], w1_ref[0], preferred_element_type=jnp.float32)
        h = jax.nn.gelu(h)
        y = jnp.dot(h, w2_ref[0], preferred_element_type=jnp.float32)
        y_ref[...] = y * wd_ref[:, :1]


def _grouped_mlp(sa, xd, W1, W2, wd):
    grid_spec = pltpu.PrefetchScalarGridSpec(
        num_scalar_prefetch=1,
        grid=(T,),
        in_specs=[
            pl.BlockSpec((B, D), lambda i, t: (t[i, 2], 0)),
            pl.BlockSpec((1, D, F), lambda i, t: (t[i, 0], 0, 0)),
            pl.BlockSpec((1, F, D), lambda i, t: (t[i, 0], 0, 0)),
            pl.BlockSpec((B, _LANES), lambda i, t: (t[i, 2], 0)),
        ],
        out_specs=pl.BlockSpec((B, D), lambda i, t: (t[i, 2], 0)),
    )
    return pl.pallas_call(
        _mlp_body,
        grid_spec=grid_spec,
        out_shape=jax.ShapeDtypeStruct((R, D), jnp.float32),
    )(sa, xd, W1, W2, wd)


_MESH = plsc.VectorSubcoreMesh(core_axis_name="c", subcore_axis_name="s")
_NW = 32
_TPW = N // _NW
_DCH = 32
_DNCH = _TPW // _DCH
_CCH = 16
_CNCH = _TPW // _CCH


@functools.partial(
    pl.kernel,
    mesh=_MESH,
    out_type=[
        jax.ShapeDtypeStruct((R, D), jnp.float32),
        jax.ShapeDtypeStruct((R, _LANES), jnp.float32),
    ],
    scratch_types=[
        pltpu.VMEM((2, _DCH, D), jnp.float32),
        pltpu.VMEM((_TPW, _LANES), jnp.float32),
        pltpu.VMEM((_TPW, _LANES), jnp.float32),
        pltpu.VMEM((_DNCH, _DCH), jnp.int32),
        pltpu.VMEM((_DNCH, _DCH), jnp.int32),
        pltpu.SemaphoreType.DMA,
        pltpu.SemaphoreType.DMA,
        pltpu.SemaphoreType.DMA,
        pltpu.SemaphoreType.DMA,
        pltpu.SemaphoreType.DMA,
        pltpu.SemaphoreType.DMA,
        pltpu.SemaphoreType.DMA,
        pltpu.SemaphoreType.DMA,
        pltpu.SemaphoreType.DMA,
        pltpu.SemaphoreType.DMA,
    ],
)
def _dispatch_sc(x_hbm, wb0_hbm, wb1_hbm, p0_hbm, p1_hbm, xd_hbm, wd_hbm,
                 xbuf, wbuf0, wbuf1, idx0b, idx1b, ls0, ls1, *ss):
    wid = lax.axis_index("s") * 2 + lax.axis_index("c")
    tok0 = wid * _TPW
    pltpu.sync_copy(wb0_hbm.at[pl.ds(tok0, _TPW)], wbuf0)
    pltpu.sync_copy(wb1_hbm.at[pl.ds(tok0, _TPW)], wbuf1)
    pltpu.sync_copy(p0_hbm.at[pl.ds(wid * _DNCH, _DNCH)], idx0b)
    pltpu.sync_copy(p1_hbm.at[pl.ds(wid * _DNCH, _DNCH)], idx1b)
    lsems = (ls0, ls1)
    loads = [None, None]
    scat = [None] * _DNCH
    loads[0] = pltpu.async_copy(x_hbm.at[pl.ds(tok0, _DCH)], xbuf.at[0], lsems[0])
    for cc in range(_DNCH):
        slot = cc % 2
        loads[slot].wait()
        sg = ss[4 * slot:4 * slot + 4]
        scat[cc] = [
            pltpu.async_copy(xbuf.at[slot], xd_hbm.at[idx0b.at[cc]], sg[0]),
            pltpu.async_copy(xbuf.at[slot], xd_hbm.at[idx1b.at[cc]], sg[1]),
            pltpu.async_copy(wbuf0.at[pl.ds(cc * _DCH, _DCH)],
                             wd_hbm.at[idx0b.at[cc]], sg[2]),
            pltpu.async_copy(wbuf1.at[pl.ds(cc * _DCH, _DCH)],
                             wd_hbm.at[idx1b.at[cc]], sg[3]),
        ]
        if cc + 1 < _DNCH:
            if cc >= 1:
                for c in scat[cc - 1]:
                    c.wait()
            loads[1 - slot] = pltpu.async_copy(
                x_hbm.at[pl.ds(tok0 + (cc + 1) * _DCH, _DCH)],
                xbuf.at[1 - slot], lsems[1 - slot])
    for cc in (_DNCH - 2, _DNCH - 1):
        for c in scat[cc]:
            c.wait()


@functools.partial(
    pl.kernel,
    mesh=_MESH,
    out_type=jax.ShapeDtypeStruct((N, D), jnp.float32),
    scratch_types=[
        pltpu.VMEM((2, _CCH, D), jnp.float32),
        pltpu.VMEM((2, _CCH, D), jnp.float32),
        pltpu.VMEM((2, _CCH, D), jnp.float32),
        pltpu.VMEM((_CNCH, _CCH), jnp.int32),
        pltpu.VMEM((_CNCH, _CCH), jnp.int32),
        pltpu.SemaphoreType.DMA,
        pltpu.SemaphoreType.DMA,
        pltpu.SemaphoreType.DMA,
        pltpu.SemaphoreType.DMA,
        pltpu.SemaphoreType.DMA,
        pltpu.SemaphoreType.DMA,
    ],
)
def _combine_sc(yd_hbm, p0_hbm, p1_hbm, out_hbm,
                b0, b1, ob, i0b, i1b, g0, g1, g2, g3, ws0, ws1):
    wid = lax.axis_index("s") * 2 + lax.axis_index("c")
    tok0 = wid * _TPW
    pltpu.sync_copy(p0_hbm.at[pl.ds(wid * _CNCH, _CNCH)], i0b)
    pltpu.sync_copy(p1_hbm.at[pl.ds(wid * _CNCH, _CNCH)], i1b)
    gs = ((g0, g1), (g2, g3))
    wsems = (ws0, ws1)
    gat = [None] * _CNCH
    wr = [None] * _CNCH
    gat[0] = (pltpu.async_copy(yd_hbm.at[i0b.at[0]], b0.at[0], gs[0][0]),
              pltpu.async_copy(yd_hbm.at[i1b.at[0]], b1.at[0], gs[0][1]))
    for cc in range(_CNCH):
        slot = cc % 2
        for c in gat[cc]:
            c.wait()
        if cc + 1 < _CNCH:
            gat[cc + 1] = (
                pltpu.async_copy(yd_hbm.at[i0b.at[cc + 1]], b0.at[1 - slot],
                                 gs[1 - slot][0]),
                pltpu.async_copy(yd_hbm.at[i1b.at[cc + 1]], b1.at[1 - slot],
                                 gs[1 - slot][1]),
            )
        if cc >= 2:
            wr[cc - 2].wait()

        def row(r, carry):
            for c in range(D // 16):
                sl = pl.ds(c * 16, 16)
                ob[slot, r, sl] = b0[slot, r, sl] + b1[slot, r, sl]
            return carry

        lax.fori_loop(0, _CCH, row, 0)
        wr[cc] = pltpu.async_copy(
            ob.at[slot], out_hbm.at[pl.ds(tok0 + cc * _CCH, _CCH)], wsems[slot])
    wr[_CNCH - 2].wait()
    wr[_CNCH - 1].wait()


def kernel(inputs, Wg, W1, W2):
    sel, wb0, wb1 = _gate(inputs, Wg)
    pos, sa = _routing(sel)
    p2 = pos.reshape(N, K)
    p0, p1 = p2[:, 0], p2[:, 1]

    xd, wd = _dispatch_sc(inputs, wb0, wb1,
                          p0.reshape(N // _DCH, _DCH), p1.reshape(N // _DCH, _DCH))

    yd = _grouped_mlp(sa, xd, W1, W2, wd)

    return _combine_sc(yd, p0.reshape(N // _CCH, _CCH), p1.reshape(N // _CCH, _CCH))


# speedup vs baseline: 1.1599x; 1.1599x over previous
import functools

import jax
import jax.numpy as jnp
from jax import lax
from jax.experimental import pallas as pl
from jax.experimental.pallas import tpu as pltpu
from jax.experimental.pallas import tpu_sc as plsc

E = 8
K = 2
D = 1024
F = 2048
N = 4096
S = N * K
B = 1024
T = (S + E * (B - 1) + B - 1) // B
R = T * B

_NEG = -1e30
_GATE_BN = 512
_LANES = 128


def _gate_body(x_ref, wg_ref, r2_ref, w0_ref, w1_ref, meta_ref, carry_ref):
    i = pl.program_id(0)

    @pl.when(i == 0)
    def _():
        carry_ref[...] = jnp.zeros_like(carry_ref)

    logits = jnp.dot(x_ref[...], wg_ref[...], preferred_element_type=jnp.float32)
    col = lax.broadcasted_iota(jnp.int32, logits.shape, 1)
    logits = jnp.where(col < E, logits, _NEG)
    m1 = jnp.max(logits, axis=1, keepdims=True)
    a1 = jnp.min(jnp.where(logits == m1, col, _LANES), axis=1, keepdims=True)
    l2 = jnp.where(col == a1, _NEG, logits)
    m2 = jnp.max(l2, axis=1, keepdims=True)
    a2 = jnp.min(jnp.where(l2 == m2, col, _LANES), axis=1, keepdims=True)
    e2 = jnp.exp(m2 - m1)
    w0 = 1.0 / (1.0 + e2)
    w0_ref[...] = jnp.broadcast_to(w0, w0_ref.shape)
    w1_ref[...] = jnp.broadcast_to(1.0 - w0, w1_ref.shape)

    oh0 = (col == a1).astype(jnp.float32)
    oh1 = (col == a2).astype(jnp.float32)
    rr = lax.broadcasted_iota(jnp.int32, (_GATE_BN, _GATE_BN), 0)
    cc = lax.broadcasted_iota(jnp.int32, (_GATE_BN, _GATE_BN), 1)
    ltri = (rr >= cc).astype(jnp.float32)
    c = jnp.dot(ltri, oh0 + oh1, preferred_element_type=jnp.float32)
    c = c + carry_ref[0:1, :]
    carry_ref[0:1, :] = c[_GATE_BN - 1:_GATE_BN, :]
    meta_ref[...] = jnp.broadcast_to(c[_GATE_BN - 1:_GATE_BN, :], meta_ref.shape)
    rank0 = jnp.sum((c - oh1) * oh0, axis=1, keepdims=True) - 1.0
    rank1 = jnp.sum(c * oh1, axis=1, keepdims=True) - 1.0
    r2_ref[...] = jnp.where(
        col == 0, rank0.astype(jnp.int32),
        jnp.where(col == 1, rank1.astype(jnp.int32),
                  jnp.where(col == 2, a1, jnp.where(col == 3, a2, 0))))


def _gate(inputs, Wg):
    wg_pad = jnp.pad(Wg, ((0, 0), (0, _LANES - E)))
    r2, wb0, wb1, meta = pl.pallas_call(
        _gate_body,
        grid=(N // _GATE_BN,),
        in_specs=[
            pl.BlockSpec((_GATE_BN, D), lambda i: (i, 0)),
            pl.BlockSpec((D, _LANES), lambda i: (0, 0)),
        ],
        out_specs=[
            pl.BlockSpec((_GATE_BN, _LANES), lambda i: (i, 0)),
            pl.BlockSpec((_GATE_BN, _LANES), lambda i: (i, 0)),
            pl.BlockSpec((_GATE_BN, _LANES), lambda i: (i, 0)),
            pl.BlockSpec((8, _LANES), lambda i: (0, 0)),
        ],
        out_shape=[
            jax.ShapeDtypeStruct((N, _LANES), jnp.int32),
            jax.ShapeDtypeStruct((N, _LANES), jnp.float32),
            jax.ShapeDtypeStruct((N, _LANES), jnp.float32),
            jax.ShapeDtypeStruct((8, _LANES), jnp.float32),
        ],
        scratch_shapes=[pltpu.VMEM((8, _LANES), jnp.float32)],
    )(inputs, wg_pad)
    return r2, wb0, wb1, meta


def _pos_body(r2_ref, meta_ref, pos2_ref, bnd_ref):
    sizes = meta_ref[0:1, :]
    lane = lax.broadcasted_iota(jnp.int32, (1, _LANES), 1)
    sizes = jnp.where(lane < E, sizes, 0.0)
    padded = jnp.ceil(sizes / B) * B
    rr = lax.broadcasted_iota(jnp.int32, (_LANES, _LANES), 0)
    cc = lax.broadcasted_iota(jnp.int32, (_LANES, _LANES), 1)
    ut = ((rr <= cc) & (rr < E)).astype(jnp.float32)
    bounds = jnp.dot(padded, ut, preferred_element_type=jnp.float32)
    starts = bounds - padded
    col = lax.broadcasted_iota(jnp.int32, pos2_ref.shape, 1)
    rank0 = r2_ref[:, 0:1]
    rank1 = r2_ref[:, 1:2]
    sel0 = r2_ref[:, 2:3]
    sel1 = r2_ref[:, 3:4]
    st0 = jnp.sum((col == sel0).astype(jnp.float32) * starts, axis=1, keepdims=True)
    st1 = jnp.sum((col == sel1).astype(jnp.float32) * starts, axis=1, keepdims=True)
    pos0 = rank0 + st0.astype(jnp.int32)
    pos1 = rank1 + st1.astype(jnp.int32)
    pos2_ref[...] = jnp.where(col == 0, pos0, jnp.where(col == 1, pos1, 0))
    bnd_ref[...] = jnp.broadcast_to(bounds, bnd_ref.shape).astype(jnp.int32)


def _positions(r2, meta):
    return pl.pallas_call(
        _pos_body,
        out_shape=[
            jax.ShapeDtypeStruct((N, _LANES), jnp.int32),
            jax.ShapeDtypeStruct((8, _LANES), jnp.int32),
        ],
    )(r2, meta)


def _tile_table(bounds):
    tile_first = jnp.arange(T, dtype=jnp.int32)[:, None] * B
    texp = jnp.minimum(jnp.sum((tile_first >= bounds[None, :]).astype(jnp.int32),
                               axis=1), E - 1)
    a_tiles = bounds[E - 1] // B
    ii = jnp.arange(T, dtype=jnp.int32)
    xblk = jnp.minimum(ii, a_tiles - 1)
    act = (ii < a_tiles).astype(jnp.int32)
    return jnp.stack([texp[xblk], act, xblk, jnp.zeros_like(ii)], axis=1)


def _mlp_body(sa_ref, x_ref, w1_ref, w2_ref, wd_ref, y_ref):
    i = pl.program_id(0)

    @pl.when(sa_ref[i, 1] != 0)
    def _():
        h = jnp.dot(x_ref[...], w1_ref[0], preferred_element_type=jnp.float32)
        h = jax.nn.gelu(h)
        y = jnp.dot(h, w2_ref[0], preferred_element_type=jnp.float32)
        y_ref[...] = y * wd_ref[:, :1]


def _grouped_mlp(sa, xd, W1, W2, wd):
    grid_spec = pltpu.PrefetchScalarGridSpec(
        num_scalar_prefetch=1,
        grid=(T,),
        in_specs=[
            pl.BlockSpec((B, D), lambda i, t: (t[i, 2], 0)),
            pl.BlockSpec((1, D, F), lambda i, t: (t[i, 0], 0, 0)),
            pl.BlockSpec((1, F, D), lambda i, t: (t[i, 0], 0, 0)),
            pl.BlockSpec((B, _LANES), lambda i, t: (t[i, 2], 0)),
        ],
        out_specs=pl.BlockSpec((B, D), lambda i, t: (t[i, 2], 0)),
    )
    return pl.pallas_call(
        _mlp_body,
        grid_spec=grid_spec,
        out_shape=jax.ShapeDtypeStruct((R, D), jnp.float32),
    )(sa, xd, W1, W2, wd)


_MESH = plsc.VectorSubcoreMesh(core_axis_name="c", subcore_axis_name="s")
_NW = 32
_TPW = N // _NW
_DCH = 32
_DNCH = _TPW // _DCH
_CCH = 16
_CNCH = _TPW // _CCH


@functools.partial(
    pl.kernel,
    mesh=_MESH,
    out_type=[
        jax.ShapeDtypeStruct((R, D), jnp.float32),
        jax.ShapeDtypeStruct((R, _LANES), jnp.float32),
    ],
    scratch_types=[
        pltpu.VMEM((2, _DCH, D), jnp.float32),
        pltpu.VMEM((_TPW, _LANES), jnp.float32),
        pltpu.VMEM((_TPW, _LANES), jnp.float32),
        pltpu.VMEM((_DNCH, _DCH), jnp.int32),
        pltpu.VMEM((_DNCH, _DCH), jnp.int32),
        pltpu.SemaphoreType.DMA,
        pltpu.SemaphoreType.DMA,
        pltpu.SemaphoreType.DMA,
        pltpu.SemaphoreType.DMA,
        pltpu.SemaphoreType.DMA,
        pltpu.SemaphoreType.DMA,
        pltpu.SemaphoreType.DMA,
        pltpu.SemaphoreType.DMA,
        pltpu.SemaphoreType.DMA,
        pltpu.SemaphoreType.DMA,
    ],
)
def _dispatch_sc(x_hbm, wb0_hbm, wb1_hbm, p0_hbm, p1_hbm, xd_hbm, wd_hbm,
                 xbuf, wbuf0, wbuf1, idx0b, idx1b, ls0, ls1, *ss):
    wid = lax.axis_index("s") * 2 + lax.axis_index("c")
    tok0 = wid * _TPW
    pltpu.sync_copy(wb0_hbm.at[pl.ds(tok0, _TPW)], wbuf0)
    pltpu.sync_copy(wb1_hbm.at[pl.ds(tok0, _TPW)], wbuf1)
    pltpu.sync_copy(p0_hbm.at[pl.ds(wid * _DNCH, _DNCH)], idx0b)
    pltpu.sync_copy(p1_hbm.at[pl.ds(wid * _DNCH, _DNCH)], idx1b)
    lsems = (ls0, ls1)
    loads = [None, None]
    scat = [None] * _DNCH
    loads[0] = pltpu.async_copy(x_hbm.at[pl.ds(tok0, _DCH)], xbuf.at[0], lsems[0])
    for cc in range(_DNCH):
        slot = cc % 2
        loads[slot].wait()
        sg = ss[4 * slot:4 * slot + 4]
        scat[cc] = [
            pltpu.async_copy(xbuf.at[slot], xd_hbm.at[idx0b.at[cc]], sg[0]),
            pltpu.async_copy(xbuf.at[slot], xd_hbm.at[idx1b.at[cc]], sg[1]),
            pltpu.async_copy(wbuf0.at[pl.ds(cc * _DCH, _DCH)],
                             wd_hbm.at[idx0b.at[cc]], sg[2]),
            pltpu.async_copy(wbuf1.at[pl.ds(cc * _DCH, _DCH)],
                             wd_hbm.at[idx1b.at[cc]], sg[3]),
        ]
        if cc + 1 < _DNCH:
            if cc >= 1:
                for c in scat[cc - 1]:
                    c.wait()
            loads[1 - slot] = pltpu.async_copy(
                x_hbm.at[pl.ds(tok0 + (cc + 1) * _DCH, _DCH)],
                xbuf.at[1 - slot], lsems[1 - slot])
    for cc in (_DNCH - 2, _DNCH - 1):
        for c in scat[cc]:
            c.wait()


@functools.partial(
    pl.kernel,
    mesh=_MESH,
    out_type=jax.ShapeDtypeStruct((N, D), jnp.float32),
    scratch_types=[
        pltpu.VMEM((2, _CCH, D), jnp.float32),
        pltpu.VMEM((2, _CCH, D), jnp.float32),
        pltpu.VMEM((2, _CCH, D), jnp.float32),
        pltpu.VMEM((_CNCH, _CCH), jnp.int32),
        pltpu.VMEM((_CNCH, _CCH), jnp.int32),
        pltpu.SemaphoreType.DMA,
        pltpu.SemaphoreType.DMA,
        pltpu.SemaphoreType.DMA,
        pltpu.SemaphoreType.DMA,
        pltpu.SemaphoreType.DMA,
        pltpu.SemaphoreType.DMA,
    ],
)
def _combine_sc(yd_hbm, p0_hbm, p1_hbm, out_hbm,
                b0, b1, ob, i0b, i1b, g0, g1, g2, g3, ws0, ws1):
    wid = lax.axis_index("s") * 2 + lax.axis_index("c")
    tok0 = wid * _TPW
    pltpu.sync_copy(p0_hbm.at[pl.ds(wid * _CNCH, _CNCH)], i0b)
    pltpu.sync_copy(p1_hbm.at[pl.ds(wid * _CNCH, _CNCH)], i1b)
    gs = ((g0, g1), (g2, g3))
    wsems = (ws0, ws1)
    gat = [None] * _CNCH
    wr = [None] * _CNCH
    gat[0] = (pltpu.async_copy(yd_hbm.at[i0b.at[0]], b0.at[0], gs[0][0]),
              pltpu.async_copy(yd_hbm.at[i1b.at[0]], b1.at[0], gs[0][1]))
    for cc in range(_CNCH):
        slot = cc % 2
        for c in gat[cc]:
            c.wait()
        if cc + 1 < _CNCH:
            gat[cc + 1] = (
                pltpu.async_copy(yd_hbm.at[i0b.at[cc + 1]], b0.at[1 - slot],
                                 gs[1 - slot][0]),
                pltpu.async_copy(yd_hbm.at[i1b.at[cc + 1]], b1.at[1 - slot],
                                 gs[1 - slot][1]),
            )
        if cc >= 2:
            wr[cc - 2].wait()

        def row(r, carry):
            for c in range(D // 16):
                sl = pl.ds(c * 16, 16)
                ob[slot, r, sl] = b0[slot, r, sl] + b1[slot, r, sl]
            return carry

        lax.fori_loop(0, _CCH, row, 0)
        wr[cc] = pltpu.async_copy(
            ob.at[slot], out_hbm.at[pl.ds(tok0 + cc * _CCH, _CCH)], wsems[slot])
    wr[_CNCH - 2].wait()
    wr[_CNCH - 1].wait()


def kernel(inputs, Wg, W1, W2):
    r2, wb0, wb1, meta = _gate(inputs, Wg)
    pos2, bnd = _positions(r2, meta)
    sa = _tile_table(bnd[0, :E])
    p0, p1 = pos2[:, 0], pos2[:, 1]

    xd, wd = _dispatch_sc(inputs, wb0, wb1,
                          p0.reshape(N // _DCH, _DCH), p1.reshape(N // _DCH, _DCH))

    yd = _grouped_mlp(sa, xd, W1, W2, wd)

    return _combine_sc(yd, p0.reshape(N // _CCH, _CCH), p1.reshape(N // _CCH, _CCH))
